# Initial kernel scaffold; baseline (speedup 1.0000x reference)
import jax
import jax.numpy as jnp
from jax.experimental import pallas as pl


def kernel(x, adj, sub_fea, W1, b1, W2, b2, fc1_w, fc1_b, att_W, att_b, att_a):
    raise NotImplementedError("write your pallas kernel here")


# speedup vs baseline: 5.4878x; 5.4878x over previous
import functools

import jax
import jax.numpy as jnp
from jax import lax
from jax.experimental import pallas as pl
from jax.experimental.pallas import tpu as pltpu
from jax.experimental.pallas import tpu_sc as plsc

N = 10000
E = 320000
NHID = 64
NCLASS = 16

NC = 2
NS = 16
NW = NC * NS
EPW = E // NW
CH = 80
STEPS = EPW // CH


def _make_spmm(feat):
    mesh = plsc.VectorSubcoreMesh(core_axis_name="c", subcore_axis_name="s")

    @functools.partial(
        pl.kernel,
        mesh=mesh,
        out_type=jax.ShapeDtypeStruct((NC, N, feat), jnp.float32),
        scratch_types=[
            pltpu.VMEM((CH,), jnp.int32),
            pltpu.VMEM((CH,), jnp.int32),
            pltpu.VMEM((CH, feat), jnp.float32),
            pltpu.VMEM_SHARED((N, feat), jnp.float32),
            pltpu.SemaphoreType.DMA,
        ],
        compiler_params=pltpu.CompilerParams(use_tc_tiling_on_sc=False),
    )
    def spmm(sup_hbm, src_hbm, dst_hbm, zero_hbm, out_hbm,
             src_v, dst_v, rows_v, acc_sh, sem):
        c = lax.axis_index("c")
        s = lax.axis_index("s")

        @pl.when(s == 0)
        def _zero():
            pltpu.sync_copy(zero_hbm, acc_sh)

        plsc.subcore_barrier()

        base = (s * NC + c) * EPW

        def step(i, carry):
            off = base + i * CH
            pltpu.sync_copy(src_hbm.at[pl.ds(off, CH)], src_v)
            pltpu.sync_copy(dst_hbm.at[pl.ds(off, CH)], dst_v)
            pltpu.async_copy(sup_hbm.at[src_v], rows_v, sem).wait()
            pltpu.sync_copy(rows_v, acc_sh.at[dst_v], add=True)
            return carry

        lax.fori_loop(0, STEPS, step, 0)
        plsc.subcore_barrier()

        @pl.when(s == 0)
        def _flush():
            pltpu.sync_copy(acc_sh, out_hbm.at[c])

    return spmm


_spmm64 = _make_spmm(NHID)
_spmm16 = _make_spmm(NCLASS)


def _tc_matmul(x, w):
    m, _ = x.shape
    n = w.shape[1]

    def body(x_ref, w_ref, o_ref):
        o_ref[...] = jnp.dot(x_ref[...], w_ref[...],
                             preferred_element_type=jnp.float32,
                             precision=lax.Precision.HIGHEST)

    return pl.pallas_call(
        body,
        out_shape=jax.ShapeDtypeStruct((m, n), jnp.float32),
    )(x, w)


def _tc_layer2(p, b1, w2):
    def body(p_ref, b_ref, w_ref, o_ref):
        h = jnp.maximum(p_ref[0] + p_ref[1] + b_ref[...], 0.0)
        o_ref[...] = jnp.dot(h, w_ref[...],
                             preferred_element_type=jnp.float32,
                             precision=lax.Precision.HIGHEST)

    return pl.pallas_call(
        body,
        out_shape=jax.ShapeDtypeStruct((N, NCLASS), jnp.float32),
    )(p, b1, w2)


def _tc_tail(p, b2, sub_fea, fc1_wT, fc1_b, att_W, att_b, att_a):
    def body(p_ref, b2_ref, sub_ref, fwT_ref, fb_ref, aW_ref, ab_ref,
             aa_ref, o_ref):
        h = jnp.maximum(p_ref[0] + p_ref[1] + b2_ref[...], 0.0)
        mean = jnp.sum(h, axis=0, keepdims=True) * (1.0 / N)
        pooled = 1.0507009873554805 * jnp.where(
            mean > 0, mean, 1.6732632423543772 * (jnp.exp(mean) - 1.0))
        x_ext = jnp.dot(sub_ref[...], fwT_ref[...],
                        preferred_element_type=jnp.float32,
                        precision=lax.Precision.HIGHEST) + fb_ref[...]
        xc = jnp.concatenate([pooled, x_ext], axis=1)
        heads = []
        for hh in range(4):
            heads.append(jnp.dot(xc, aW_ref[hh],
                                 preferred_element_type=jnp.float32,
                                 precision=lax.Precision.HIGHEST)
                         + ab_ref[hh:hh + 1])
        hm = jnp.concatenate(heads, axis=0)
        scores = jnp.sum(hm * aa_ref[...], axis=1, keepdims=True)
        mx = jnp.max(scores, axis=0, keepdims=True)
        ex = jnp.exp(scores - mx)
        alpha = ex / jnp.sum(ex, axis=0, keepdims=True)
        out = jnp.sum(alpha * hm, axis=0, keepdims=True)
        m2 = jnp.max(out, axis=1, keepdims=True)
        lse = jnp.log(jnp.sum(jnp.exp(out - m2), axis=1, keepdims=True)) + m2
        o_ref[...] = out - lse

    return pl.pallas_call(
        body,
        out_shape=jax.ShapeDtypeStruct((1, NCLASS), jnp.float32),
    )(p, b2, sub_fea, fc1_wT, fc1_b, att_W, att_b, att_a)


def kernel(x, adj, sub_fea, W1, b1, W2, b2, fc1_w, fc1_b, att_W, att_b, att_a):
    src = adj[0]
    dst = adj[1]
    zeros64 = jnp.zeros((N, NHID), jnp.float32)
    zeros16 = jnp.zeros((N, NCLASS), jnp.float32)

    support1 = _tc_matmul(x, W1)
    p1 = _spmm64(support1, src, dst, zeros64)
    support2 = _tc_layer2(p1, b1.reshape(1, -1), W2)
    p2 = _spmm16(support2, src, dst, zeros16)
    return _tc_tail(p2, b2.reshape(1, -1), sub_fea, fc1_w.T,
                    fc1_b.reshape(1, -1), att_W, att_b, att_a)


# speedup vs baseline: 12.2327x; 2.2291x over previous
import functools

import jax
import jax.numpy as jnp
from jax import lax
from jax.experimental import pallas as pl
from jax.experimental.pallas import tpu as pltpu
from jax.experimental.pallas import tpu_sc as plsc

N = 10000
E = 320000
NHID = 64
NCLASS = 16

NC = 2
NS = 16
NW = NC * NS
CH = 128
GPW = 80
CHR = GPW + 1
NCHUNK = NW * GPW
EPAD = (NCHUNK + 1) * CH - E
NPAD = 10016
RPT = NPAD // NS


def _make_spmm(feat):
    mesh = plsc.VectorSubcoreMesh(core_axis_name="c", subcore_axis_name="s")

    @functools.partial(
        pl.kernel,
        mesh=mesh,
        out_type=jax.ShapeDtypeStruct((NC, NPAD, feat), jnp.float32),
        scratch_types=[
            pltpu.VMEM((CHR, CH), jnp.int32),
            pltpu.VMEM((CHR, CH), jnp.int32),
            pltpu.VMEM((2, CH, feat), jnp.float32),
            pltpu.VMEM_SHARED((NPAD, feat), jnp.float32),
            pltpu.SemaphoreType.DMA,
            pltpu.SemaphoreType.DMA,
        ],
        compiler_params=pltpu.CompilerParams(use_tc_tiling_on_sc=False),
    )
    def spmm(sup_hbm, srcp_hbm, dstp_hbm, zero_hbm, out_hbm,
             src_v, dst_v, rows_v, acc_sh, gsem0, gsem1):
        c = lax.axis_index("c")
        s = lax.axis_index("s")
        w = s * NC + c
        first = w * GPW

        pltpu.sync_copy(zero_hbm.at[pl.ds(s * RPT, RPT)],
                        acc_sh.at[pl.ds(s * RPT, RPT)])
        pltpu.sync_copy(srcp_hbm.at[pl.ds(first, CHR)], src_v)
        pltpu.sync_copy(dstp_hbm.at[pl.ds(first, CHR)], dst_v)
        plsc.subcore_barrier()

        def fire(i, b, sem):
            pltpu.async_copy(sup_hbm.at[src_v.at[i]], rows_v.at[b], sem)

        def wait_g(b, sem):
            pltpu.make_async_copy(sup_hbm.at[src_v.at[0]], rows_v.at[b],
                                  sem).wait()

        def scat(i, b):
            pltpu.sync_copy(rows_v.at[b], acc_sh.at[dst_v.at[i]], add=True)

        fire(0, 0, gsem0)

        def body(j, carry):
            i0 = 2 * j
            i1 = i0 + 1
            wait_g(0, gsem0)
            fire(i1, 1, gsem1)
            scat(i0, 0)
            wait_g(1, gsem1)
            fire(i1 + 1, 0, gsem0)
            scat(i1, 1)
            return carry

        lax.fori_loop(0, GPW // 2, body, 0)
        wait_g(0, gsem0)
        plsc.subcore_barrier()

        pltpu.sync_copy(acc_sh.at[pl.ds(s * RPT, RPT)],
                        out_hbm.at[c].at[pl.ds(s * RPT, RPT)])

    return spmm


_spmm64 = _make_spmm(NHID)
_spmm16 = _make_spmm(NCLASS)


def _tc_matmul(x, w):
    m, _ = x.shape
    n = w.shape[1]

    def body(x_ref, w_ref, o_ref):
        o_ref[...] = jnp.dot(x_ref[...], w_ref[...],
                             preferred_element_type=jnp.float32,
                             precision=lax.Precision.HIGHEST)

    return pl.pallas_call(
        body,
        out_shape=jax.ShapeDtypeStruct((m, n), jnp.float32),
    )(x, w)


def _tc_layer2(p, b1, w2):
    def body(p_ref, b_ref, w_ref, o_ref):
        h = jnp.maximum(p_ref[0] + p_ref[1] + b_ref[...], 0.0)
        o_ref[...] = jnp.dot(h, w_ref[...],
                             preferred_element_type=jnp.float32,
                             precision=lax.Precision.HIGHEST)

    return pl.pallas_call(
        body,
        out_shape=jax.ShapeDtypeStruct((NPAD, NCLASS), jnp.float32),
    )(p, b1, w2)


def _tc_tail(p, b2, sub_fea, fc1_wT, fc1_b, att_W, att_b, att_a):
    def body(p_ref, b2_ref, sub_ref, fwT_ref, fb_ref, aW_ref, ab_ref,
             aa_ref, o_ref):
        h = jnp.maximum(p_ref[0, :N, :] + p_ref[1, :N, :] + b2_ref[...], 0.0)
        mean = jnp.sum(h, axis=0, keepdims=True) * (1.0 / N)
        pooled = 1.0507009873554805 * jnp.where(
            mean > 0, mean, 1.6732632423543772 * (jnp.exp(mean) - 1.0))
        x_ext = jnp.dot(sub_ref[...], fwT_ref[...],
                        preferred_element_type=jnp.float32,
                        precision=lax.Precision.HIGHEST) + fb_ref[...]
        xc = jnp.concatenate([pooled, x_ext], axis=1)
        heads = []
        for hh in range(4):
            heads.append(jnp.dot(xc, aW_ref[hh],
                                 preferred_element_type=jnp.float32,
                                 precision=lax.Precision.HIGHEST)
                         + ab_ref[hh:hh + 1])
        hm = jnp.concatenate(heads, axis=0)
        scores = jnp.sum(hm * aa_ref[...], axis=1, keepdims=True)
        mx = jnp.max(scores, axis=0, keepdims=True)
        ex = jnp.exp(scores - mx)
        alpha = ex / jnp.sum(ex, axis=0, keepdims=True)
        out = jnp.sum(alpha * hm, axis=0, keepdims=True)
        m2 = jnp.max(out, axis=1, keepdims=True)
        lse = jnp.log(jnp.sum(jnp.exp(out - m2), axis=1, keepdims=True)) + m2
        o_ref[...] = out - lse

    return pl.pallas_call(
        body,
        out_shape=jax.ShapeDtypeStruct((1, NCLASS), jnp.float32),
    )(p, b2, sub_fea, fc1_wT, fc1_b, att_W, att_b, att_a)


def kernel(x, adj, sub_fea, W1, b1, W2, b2, fc1_w, fc1_b, att_W, att_b, att_a):
    pad_src = (jnp.arange(EPAD, dtype=jnp.int32) * 997) % N
    pad_dst = N + (jnp.arange(EPAD, dtype=jnp.int32) % 8)
    srcp = jnp.concatenate([adj[0], pad_src]).reshape(NCHUNK + 1, CH)
    dstp = jnp.concatenate([adj[1], pad_dst]).reshape(NCHUNK + 1, CH)
    zeros64 = jnp.zeros((NPAD, NHID), jnp.float32)
    zeros16 = jnp.zeros((NPAD, NCLASS), jnp.float32)

    support1 = _tc_matmul(x, W1)
    p1 = _spmm64(support1, srcp, dstp, zeros64)
    support2 = _tc_layer2(p1, b1.reshape(1, -1), W2)
    p2 = _spmm16(support2, srcp, dstp, zeros16)
    return _tc_tail(p2, b2.reshape(1, -1), sub_fea, fc1_w.T,
                    fc1_b.reshape(1, -1), att_W, att_b, att_a)


# speedup vs baseline: 16.1233x; 1.3181x over previous
import functools

import jax
import jax.numpy as jnp
from jax import lax
from jax.experimental import pallas as pl
from jax.experimental.pallas import tpu as pltpu
from jax.experimental.pallas import tpu_sc as plsc

N = 10000
E = 320000
NHID = 64
NCLASS = 16

NC = 2
NS = 16
NW = NC * NS
CH = 128
GPW = 80
CHR = GPW + 2
NCHUNK = NW * GPW + 2
EPAD = NCHUNK * CH - E
NPAD = 10016
RPT = NPAD // NS


def _make_spmm(feat):
    mesh = plsc.VectorSubcoreMesh(core_axis_name="c", subcore_axis_name="s")

    @functools.partial(
        pl.kernel,
        mesh=mesh,
        out_type=jax.ShapeDtypeStruct((NC, NPAD, feat), jnp.float32),
        scratch_types=[
            pltpu.VMEM((CHR, CH), jnp.int32),
            pltpu.VMEM((CHR, CH), jnp.int32),
            pltpu.VMEM((4, CH, feat), jnp.float32),
            pltpu.VMEM_SHARED((NPAD, feat), jnp.float32),
            pltpu.SemaphoreType.DMA,
            pltpu.SemaphoreType.DMA,
            pltpu.SemaphoreType.DMA,
            pltpu.SemaphoreType.DMA,
            pltpu.SemaphoreType.DMA,
            pltpu.SemaphoreType.DMA,
            pltpu.SemaphoreType.DMA,
            pltpu.SemaphoreType.DMA,
        ],
        compiler_params=pltpu.CompilerParams(use_tc_tiling_on_sc=False),
    )
    def spmm(sup_hbm, adjp_hbm, zero_hbm, out_hbm,
             src_v, dst_v, rows_v, acc_sh,
             g0, g1, g2, g3, s0, s1, s2, s3):
        gsems = (g0, g1, g2, g3)
        ssems = (s0, s1, s2, s3)
        c = lax.axis_index("c")
        s = lax.axis_index("s")
        w = s * NC + c
        first = w * GPW

        pltpu.sync_copy(zero_hbm.at[pl.ds(s * RPT, RPT)],
                        acc_sh.at[pl.ds(s * RPT, RPT)])
        pltpu.sync_copy(adjp_hbm.at[0, pl.ds(first, CHR)], src_v)
        pltpu.sync_copy(adjp_hbm.at[1, pl.ds(first, CHR)], dst_v)
        plsc.subcore_barrier()

        def fire_g(i, b):
            pltpu.async_copy(sup_hbm.at[src_v.at[i]], rows_v.at[b], gsems[b])

        def wait_g(b):
            pltpu.make_async_copy(sup_hbm.at[src_v.at[0]], rows_v.at[b],
                                  gsems[b]).wait()

        def fire_s(i, b):
            pltpu.async_copy(rows_v.at[b], acc_sh.at[dst_v.at[i]], ssems[b],
                             add=True)

        def wait_s(b):
            pltpu.make_async_copy(rows_v.at[b], acc_sh.at[dst_v.at[0]],
                                  ssems[b]).wait()

        fire_g(0, 0)
        fire_g(1, 1)
        wait_g(0); fire_s(0, 0); fire_g(2, 2)
        wait_g(1); fire_s(1, 1); fire_g(3, 3)
        wait_g(2); fire_s(2, 2); wait_s(0); fire_g(4, 0)
        wait_g(3); fire_s(3, 3); wait_s(1); fire_g(5, 1)

        def body(j, carry):
            for k in range(4):
                i = 4 * j + 4 + k
                wait_g(k)
                fire_s(i, k)
                wait_s((k + 2) % 4)
                fire_g(i + 2, (k + 2) % 4)
            return carry

        lax.fori_loop(0, (GPW - 4) // 4, body, 0)
        wait_g(0)
        wait_g(1)
        wait_s(2)
        wait_s(3)
        plsc.subcore_barrier()

        pltpu.sync_copy(acc_sh.at[pl.ds(s * RPT, RPT)],
                        out_hbm.at[c].at[pl.ds(s * RPT, RPT)])

    return spmm


_spmm64 = _make_spmm(NHID)
_spmm16 = _make_spmm(NCLASS)


def _tc_matmul(x, w):
    m, _ = x.shape
    n = w.shape[1]

    def body(x_ref, w_ref, o_ref):
        o_ref[...] = jnp.dot(x_ref[...], w_ref[...],
                             preferred_element_type=jnp.float32,
                             precision=lax.Precision.HIGHEST)

    return pl.pallas_call(
        body,
        out_shape=jax.ShapeDtypeStruct((m, n), jnp.float32),
    )(x, w)


def _tc_layer2(p, b1, w2):
    def body(p_ref, b_ref, w_ref, o_ref):
        h = jnp.maximum(p_ref[0] + p_ref[1] + b_ref[...], 0.0)
        o_ref[...] = jnp.dot(h, w_ref[...],
                             preferred_element_type=jnp.float32,
                             precision=lax.Precision.HIGHEST)

    return pl.pallas_call(
        body,
        out_shape=jax.ShapeDtypeStruct((NPAD, NCLASS), jnp.float32),
    )(p, b1, w2)


def _tc_tail(p, b2, sub_fea, fc1_wT, fc1_b, att_W, att_b, att_a):
    def body(p_ref, b2_ref, sub_ref, fwT_ref, fb_ref, aW_ref, ab_ref,
             aa_ref, o_ref):
        h = jnp.maximum(p_ref[0, :N, :] + p_ref[1, :N, :] + b2_ref[...], 0.0)
        mean = jnp.sum(h, axis=0, keepdims=True) * (1.0 / N)
        pooled = 1.0507009873554805 * jnp.where(
            mean > 0, mean, 1.6732632423543772 * (jnp.exp(mean) - 1.0))
        x_ext = jnp.dot(sub_ref[...], fwT_ref[...],
                        preferred_element_type=jnp.float32,
                        precision=lax.Precision.HIGHEST) + fb_ref[...]
        xc = jnp.concatenate([pooled, x_ext], axis=1)
        heads = []
        for hh in range(4):
            heads.append(jnp.dot(xc, aW_ref[hh],
                                 preferred_element_type=jnp.float32,
                                 precision=lax.Precision.HIGHEST)
                         + ab_ref[hh:hh + 1])
        hm = jnp.concatenate(heads, axis=0)
        scores = jnp.sum(hm * aa_ref[...], axis=1, keepdims=True)
        mx = jnp.max(scores, axis=0, keepdims=True)
        ex = jnp.exp(scores - mx)
        alpha = ex / jnp.sum(ex, axis=0, keepdims=True)
        out = jnp.sum(alpha * hm, axis=0, keepdims=True)
        m2 = jnp.max(out, axis=1, keepdims=True)
        lse = jnp.log(jnp.sum(jnp.exp(out - m2), axis=1, keepdims=True)) + m2
        o_ref[...] = out - lse

    return pl.pallas_call(
        body,
        out_shape=jax.ShapeDtypeStruct((1, NCLASS), jnp.float32),
    )(p, b2, sub_fea, fc1_wT, fc1_b, att_W, att_b, att_a)


def kernel(x, adj, sub_fea, W1, b1, W2, b2, fc1_w, fc1_b, att_W, att_b, att_a):
    pad_src = (jnp.arange(EPAD, dtype=jnp.int32) * 997) % N
    pad_dst = N + (jnp.arange(EPAD, dtype=jnp.int32) % 8)
    adjp = jnp.concatenate(
        [adj, jnp.stack([pad_src, pad_dst])], axis=1).reshape(2, NCHUNK, CH)
    zeros64 = jnp.zeros((NPAD, NHID), jnp.float32)
    zeros16 = jnp.zeros((NPAD, NCLASS), jnp.float32)

    support1 = _tc_matmul(x, W1)
    p1 = _spmm64(support1, adjp, zeros64)
    support2 = _tc_layer2(p1, b1.reshape(1, -1), W2)
    p2 = _spmm16(support2, adjp, zeros16)
    return _tc_tail(p2, b2.reshape(1, -1), sub_fea, fc1_w.T,
                    fc1_b.reshape(1, -1), att_W, att_b, att_a)


# speedup vs baseline: 19.0221x; 1.1798x over previous
import functools

import jax
import jax.numpy as jnp
from jax import lax
from jax.experimental import pallas as pl
from jax.experimental.pallas import tpu as pltpu
from jax.experimental.pallas import tpu_sc as plsc

N = 10000
E = 320000
NHID = 64
NCLASS = 16

NC = 2
NS = 16
NW = NC * NS
CH = 128
GPW = 80
D = 8
CHR = GPW + D // 2
NCHUNK = NW * GPW + D // 2
EPAD = NCHUNK * CH - E
NPAD = 10016
RPT = NPAD // NS


def _make_spmm(feat):
    mesh = plsc.VectorSubcoreMesh(core_axis_name="c", subcore_axis_name="s")

    @functools.partial(
        pl.kernel,
        mesh=mesh,
        out_type=jax.ShapeDtypeStruct((NC, NPAD, feat), jnp.float32),
        scratch_types=[
            pltpu.VMEM((CHR, CH), jnp.int32),
            pltpu.VMEM((CHR, CH), jnp.int32),
            pltpu.VMEM((D, CH, feat), jnp.float32),
            pltpu.VMEM_SHARED((NPAD, feat), jnp.float32),
        ] + [pltpu.SemaphoreType.DMA] * (2 * D),
        compiler_params=pltpu.CompilerParams(use_tc_tiling_on_sc=False),
    )
    def spmm(sup_hbm, adjp_hbm, zero_hbm, out_hbm,
             src_v, dst_v, rows_v, acc_sh, *sems):
        gsems = sems[:D]
        ssems = sems[D:]
        c = lax.axis_index("c")
        s = lax.axis_index("s")
        w = s * NC + c
        first = w * GPW

        zrow = s * RPT
        pltpu.async_copy(zero_hbm.at[pl.ds(zrow, RPT)],
                         acc_sh.at[pl.ds(zrow, RPT)], ssems[0])
        pltpu.sync_copy(adjp_hbm.at[0, pl.ds(first, CHR)], src_v)
        pltpu.sync_copy(adjp_hbm.at[1, pl.ds(first, CHR)], dst_v)
        pltpu.make_async_copy(zero_hbm.at[pl.ds(zrow, RPT)],
                              acc_sh.at[pl.ds(zrow, RPT)], ssems[0]).wait()
        plsc.subcore_barrier()

        def fire_g(i, b):
            pltpu.async_copy(sup_hbm.at[src_v.at[i]], rows_v.at[b], gsems[b])

        def wait_g(b):
            pltpu.make_async_copy(sup_hbm.at[src_v.at[0]], rows_v.at[b],
                                  gsems[b]).wait()

        def fire_s(i, b):
            pltpu.async_copy(rows_v.at[b], acc_sh.at[dst_v.at[i]], ssems[b],
                             add=True)

        def wait_s(b):
            pltpu.make_async_copy(rows_v.at[b], acc_sh.at[dst_v.at[0]],
                                  ssems[b]).wait()

        for i in range(4):
            fire_g(i, i)
        for i in range(4):
            wait_g(i)
            fire_s(i, i)
            fire_g(i + 4, i + 4)
        for i in range(4, 8):
            wait_g(i)
            fire_s(i, i)
            wait_s(i - 4)
            fire_g(i + 4, i - 4)

        def body(j, carry):
            for k in range(D):
                i = D * j + D + k
                wait_g(k)
                fire_s(i, k)
                wait_s((k + 4) % D)
                fire_g(i + 4, (k + 4) % D)
            return carry

        lax.fori_loop(0, (GPW - D) // D, body, 0)
        for b in range(4):
            wait_g(b)
        for b in range(4, 8):
            wait_s(b)
        plsc.subcore_barrier()

        pltpu.sync_copy(acc_sh.at[pl.ds(zrow, RPT)],
                        out_hbm.at[c].at[pl.ds(zrow, RPT)])

    return spmm


_spmm64 = _make_spmm(NHID)
_spmm16 = _make_spmm(NCLASS)


def _tc_matmul(x, w):
    m, k = x.shape
    n = w.shape[1]
    blk = 2000

    def body(x_ref, w_ref, o_ref):
        o_ref[...] = jnp.dot(x_ref[...], w_ref[...],
                             preferred_element_type=jnp.float32)

    return pl.pallas_call(
        body,
        grid=(m // blk,),
        in_specs=[pl.BlockSpec((blk, k), lambda i: (i, 0)),
                  pl.BlockSpec((k, n), lambda i: (0, 0))],
        out_specs=pl.BlockSpec((blk, n), lambda i: (i, 0)),
        out_shape=jax.ShapeDtypeStruct((m, n), jnp.float32),
    )(x, w)


def _tc_layer2(p, b1, w2):
    blk = 2504

    def body(p_ref, b_ref, w_ref, o_ref):
        h = jnp.maximum(p_ref[0] + p_ref[1] + b_ref[...], 0.0)
        o_ref[...] = jnp.dot(h, w_ref[...],
                             preferred_element_type=jnp.float32)

    return pl.pallas_call(
        body,
        grid=(NPAD // blk,),
        in_specs=[pl.BlockSpec((2, blk, NHID), lambda i: (0, i, 0)),
                  pl.BlockSpec((1, NHID), lambda i: (0, 0)),
                  pl.BlockSpec((NHID, NCLASS), lambda i: (0, 0))],
        out_specs=pl.BlockSpec((blk, NCLASS), lambda i: (i, 0)),
        out_shape=jax.ShapeDtypeStruct((NPAD, NCLASS), jnp.float32),
    )(p, b1, w2)


def _tc_tail(p, b2, sub_fea, fc1_wT, fc1_b, att_W, att_b, att_a):
    def body(p_ref, b2_ref, sub_ref, fwT_ref, fb_ref, aW_ref, ab_ref,
             aa_ref, o_ref):
        h = jnp.maximum(p_ref[0, :N, :] + p_ref[1, :N, :] + b2_ref[...], 0.0)
        mean = jnp.sum(h, axis=0, keepdims=True) * (1.0 / N)
        pooled = 1.0507009873554805 * jnp.where(
            mean > 0, mean, 1.6732632423543772 * (jnp.exp(mean) - 1.0))
        x_ext = jnp.dot(sub_ref[...], fwT_ref[...],
                        preferred_element_type=jnp.float32) + fb_ref[...]
        xc = jnp.concatenate([pooled, x_ext], axis=1)
        heads = []
        for hh in range(4):
            heads.append(jnp.dot(xc, aW_ref[hh],
                                 preferred_element_type=jnp.float32)
                         + ab_ref[hh:hh + 1])
        hm = jnp.concatenate(heads, axis=0)
        scores = jnp.sum(hm * aa_ref[...], axis=1, keepdims=True)
        mx = jnp.max(scores, axis=0, keepdims=True)
        ex = jnp.exp(scores - mx)
        alpha = ex / jnp.sum(ex, axis=0, keepdims=True)
        out = jnp.sum(alpha * hm, axis=0, keepdims=True)
        m2 = jnp.max(out, axis=1, keepdims=True)
        lse = jnp.log(jnp.sum(jnp.exp(out - m2), axis=1, keepdims=True)) + m2
        o_ref[...] = out - lse

    return pl.pallas_call(
        body,
        out_shape=jax.ShapeDtypeStruct((1, NCLASS), jnp.float32),
    )(p, b2, sub_fea, fc1_wT, fc1_b, att_W, att_b, att_a)


def kernel(x, adj, sub_fea, W1, b1, W2, b2, fc1_w, fc1_b, att_W, att_b, att_a):
    pad_src = (jnp.arange(EPAD, dtype=jnp.int32) * 997) % N
    pad_dst = N + (jnp.arange(EPAD, dtype=jnp.int32) % 8)
    adjp = jnp.concatenate(
        [adj, jnp.stack([pad_src, pad_dst])], axis=1).reshape(2, NCHUNK, CH)
    zeros64 = jnp.zeros((NPAD, NHID), jnp.float32)
    zeros16 = jnp.zeros((NPAD, NCLASS), jnp.float32)

    support1 = _tc_matmul(x, W1)
    p1 = _spmm64(support1, adjp, zeros64)
    support2 = _tc_layer2(p1, b1.reshape(1, -1), W2)
    p2 = _spmm16(support2, adjp, zeros16)
    return _tc_tail(p2, b2.reshape(1, -1), sub_fea, fc1_w.T,
                    fc1_b.reshape(1, -1), att_W, att_b, att_a)


# speedup vs baseline: 23.0095x; 1.2096x over previous
import functools

import numpy as np

import jax
import jax.numpy as jnp
from jax import lax
from jax.experimental import pallas as pl
from jax.experimental.pallas import tpu as pltpu
from jax.experimental.pallas import tpu_sc as plsc

N = 10000
E = 320000
NFEAT = 128
NHID = 64
NCLASS = 16

NC = 2
NS = 16
NW = NC * NS
CH = 128
GPW = 80
D = 8
CHR = GPW + D // 2
NREAL = E // CH
LASTW = NREAL - (NW - 1) * GPW
NPADC = CHR - LASTW
NPAD = 10112
RPT = NPAD // NS
SRPT = N // NS
FH = 32
GF = 160
CHRF = GF + D // 2
LASTF = NREAL - (NS - 1) * GF

_PAD_NP = np.stack([
    (np.arange(NPADC * CH, dtype=np.int32) * 997) % N,
    N + (np.arange(NPADC * CH, dtype=np.int32) % 8),
]).reshape(2, NPADC, CH)

_ZERO_NP = np.zeros((NPAD, NHID), np.float32)


def _make_spmm(feat):
    mesh = plsc.VectorSubcoreMesh(core_axis_name="c", subcore_axis_name="s")

    @functools.partial(
        pl.kernel,
        mesh=mesh,
        out_type=jax.ShapeDtypeStruct((NC, NPAD, feat), jnp.float32),
        scratch_types=[
            pltpu.VMEM((CHR, CH), jnp.int32),
            pltpu.VMEM((CHR, CH), jnp.int32),
            pltpu.VMEM((D, CH, feat), jnp.float32),
            pltpu.VMEM_SHARED((NPAD, feat), jnp.float32),
            pltpu.VMEM_SHARED((N, feat), jnp.float32),
        ] + [pltpu.SemaphoreType.DMA] * (2 * D),
        compiler_params=pltpu.CompilerParams(use_tc_tiling_on_sc=False),
    )
    def spmm(sup_hbm, adj2_hbm, padc_hbm, zero_hbm, out_hbm,
             src_v, dst_v, rows_v, acc_sh, sup_sh, *sems):
        gsems = sems[:D]
        ssems = sems[D:]
        c = lax.axis_index("c")
        s = lax.axis_index("s")
        w = s * NC + c
        first = w * GPW

        zrow = s * RPT
        srow = s * SRPT
        pltpu.async_copy(zero_hbm.at[pl.ds(zrow, RPT), pl.ds(0, feat)],
                         acc_sh.at[pl.ds(zrow, RPT)], ssems[0])
        pltpu.async_copy(sup_hbm.at[pl.ds(srow, SRPT)],
                         sup_sh.at[pl.ds(srow, SRPT)], ssems[1])

        @pl.when(w < NW - 1)
        def _stage():
            pltpu.sync_copy(adj2_hbm.at[0, pl.ds(first, CHR)], src_v)
            pltpu.sync_copy(adj2_hbm.at[1, pl.ds(first, CHR)], dst_v)

        @pl.when(w == NW - 1)
        def _stage_last():
            pltpu.sync_copy(adj2_hbm.at[0, pl.ds(first, LASTW)],
                            src_v.at[pl.ds(0, LASTW)])
            pltpu.sync_copy(adj2_hbm.at[1, pl.ds(first, LASTW)],
                            dst_v.at[pl.ds(0, LASTW)])
            pltpu.sync_copy(padc_hbm.at[0], src_v.at[pl.ds(LASTW, NPADC)])
            pltpu.sync_copy(padc_hbm.at[1], dst_v.at[pl.ds(LASTW, NPADC)])

        pltpu.make_async_copy(zero_hbm.at[pl.ds(zrow, RPT), pl.ds(0, feat)],
                              acc_sh.at[pl.ds(zrow, RPT)], ssems[0]).wait()
        pltpu.make_async_copy(sup_hbm.at[pl.ds(srow, SRPT)],
                              sup_sh.at[pl.ds(srow, SRPT)], ssems[1]).wait()
        plsc.subcore_barrier()

        def fire_g(i, b):
            pltpu.async_copy(sup_sh.at[src_v.at[i]], rows_v.at[b], gsems[b])

        def wait_g(b):
            pltpu.make_async_copy(sup_sh.at[src_v.at[0]], rows_v.at[b],
                                  gsems[b]).wait()

        def fire_s(i, b):
            pltpu.async_copy(rows_v.at[b], acc_sh.at[dst_v.at[i]], ssems[b],
                             add=True)

        def wait_s(b):
            pltpu.make_async_copy(rows_v.at[b], acc_sh.at[dst_v.at[0]],
                                  ssems[b]).wait()

        for i in range(4):
            fire_g(i, i)
        for i in range(4):
            wait_g(i)
            fire_s(i, i)
            fire_g(i + 4, i + 4)
        for i in range(4, 8):
            wait_g(i)
            fire_s(i, i)
            wait_s(i - 4)
            fire_g(i + 4, i - 4)

        def body(j, carry):
            for k in range(D):
                i = D * j + D + k
                wait_g(k)
                fire_s(i, k)
                wait_s((k + 4) % D)
                fire_g(i + 4, (k + 4) % D)
            return carry

        lax.fori_loop(0, (GPW - D) // D, body, 0)
        for b in range(4):
            wait_g(b)
        for b in range(4, 8):
            wait_s(b)
        plsc.subcore_barrier()

        pltpu.sync_copy(acc_sh.at[pl.ds(zrow, RPT)],
                        out_hbm.at[c].at[pl.ds(zrow, RPT)])

    return spmm


def _make_spmm_fsplit():
    mesh = plsc.VectorSubcoreMesh(core_axis_name="c", subcore_axis_name="s")

    @functools.partial(
        pl.kernel,
        mesh=mesh,
        out_type=jax.ShapeDtypeStruct((NC, NPAD, FH), jnp.float32),
        scratch_types=[
            pltpu.VMEM((CHRF, CH), jnp.int32),
            pltpu.VMEM((CHRF, CH), jnp.int32),
            pltpu.VMEM((D, CH, FH), jnp.float32),
            pltpu.VMEM_SHARED((NPAD, FH), jnp.float32),
            pltpu.VMEM_SHARED((N, FH), jnp.float32),
        ] + [pltpu.SemaphoreType.DMA] * (2 * D),
        compiler_params=pltpu.CompilerParams(use_tc_tiling_on_sc=False),
    )
    def spmm(sup_hbm, adj2_hbm, padc_hbm, zero_hbm, out_hbm,
             src_v, dst_v, rows_v, acc_sh, sup_sh, *sems):
        gsems = sems[:D]
        ssems = sems[D:]
        c = lax.axis_index("c")
        s = lax.axis_index("s")
        first = s * GF

        zrow = s * RPT
        srow = s * SRPT
        pltpu.async_copy(zero_hbm.at[pl.ds(zrow, RPT), pl.ds(0, FH)],
                         acc_sh.at[pl.ds(zrow, RPT)], ssems[0])
        pltpu.async_copy(sup_hbm.at[pl.ds(srow, SRPT), pl.ds(c * FH, FH)],
                         sup_sh.at[pl.ds(srow, SRPT)], ssems[1])

        @pl.when(s < NS - 1)
        def _stage():
            pltpu.sync_copy(adj2_hbm.at[0, pl.ds(first, CHRF)], src_v)
            pltpu.sync_copy(adj2_hbm.at[1, pl.ds(first, CHRF)], dst_v)

        @pl.when(s == NS - 1)
        def _stage_last():
            pltpu.sync_copy(adj2_hbm.at[0, pl.ds(first, LASTF)],
                            src_v.at[pl.ds(0, LASTF)])
            pltpu.sync_copy(adj2_hbm.at[1, pl.ds(first, LASTF)],
                            dst_v.at[pl.ds(0, LASTF)])
            pltpu.sync_copy(padc_hbm.at[0], src_v.at[pl.ds(LASTF, NPADC)])
            pltpu.sync_copy(padc_hbm.at[1], dst_v.at[pl.ds(LASTF, NPADC)])

        pltpu.make_async_copy(zero_hbm.at[pl.ds(zrow, RPT), pl.ds(0, FH)],
                              acc_sh.at[pl.ds(zrow, RPT)], ssems[0]).wait()
        pltpu.make_async_copy(sup_hbm.at[pl.ds(srow, SRPT),
                                         pl.ds(c * FH, FH)],
                              sup_sh.at[pl.ds(srow, SRPT)], ssems[1]).wait()
        plsc.subcore_barrier()

        def fire_g(i, b):
            pltpu.async_copy(sup_sh.at[src_v.at[i]], rows_v.at[b], gsems[b])

        def wait_g(b):
            pltpu.make_async_copy(sup_sh.at[src_v.at[0]], rows_v.at[b],
                                  gsems[b]).wait()

        def fire_s(i, b):
            pltpu.async_copy(rows_v.at[b], acc_sh.at[dst_v.at[i]], ssems[b],
                             add=True)

        def wait_s(b):
            pltpu.make_async_copy(rows_v.at[b], acc_sh.at[dst_v.at[0]],
                                  ssems[b]).wait()

        for i in range(4):
            fire_g(i, i)
        for i in range(4):
            wait_g(i)
            fire_s(i, i)
            fire_g(i + 4, i + 4)
        for i in range(4, 8):
            wait_g(i)
            fire_s(i, i)
            wait_s(i - 4)
            fire_g(i + 4, i - 4)

        def body(j, carry):
            for k in range(D):
                i = D * j + D + k
                wait_g(k)
                fire_s(i, k)
                wait_s((k + 4) % D)
                fire_g(i + 4, (k + 4) % D)
            return carry

        lax.fori_loop(0, (GF - D) // D, body, 0)
        for b in range(4):
            wait_g(b)
        for b in range(4, 8):
            wait_s(b)
        plsc.subcore_barrier()

        pltpu.sync_copy(acc_sh.at[pl.ds(zrow, RPT)],
                        out_hbm.at[c].at[pl.ds(zrow, RPT)])

    return spmm


_spmm64 = _make_spmm_fsplit()
_spmm16 = _make_spmm(NCLASS)


def _tc_matmul(x, w):
    blk = 2000

    def body(x_ref, w_ref, o_ref):
        o_ref[...] = jnp.dot(x_ref[...], w_ref[...],
                             preferred_element_type=jnp.float32)

    return pl.pallas_call(
        body,
        grid=(N // blk,),
        in_specs=[pl.BlockSpec((blk, NFEAT), lambda i: (i, 0)),
                  pl.BlockSpec((NFEAT, NHID), lambda i: (0, 0))],
        out_specs=pl.BlockSpec((blk, NHID), lambda i: (i, 0)),
        out_shape=jax.ShapeDtypeStruct((N, NHID), jnp.float32),
    )(x, w)


def _tc_layer2(pv, b1, w2lo4, w2hi4):
    vin = NPAD * FH // 128
    blk = vin // 2

    def body(p_ref, b_ref, wlo_ref, whi_ref, o_ref):
        blo = jnp.tile(b_ref[:, 0:FH], (1, 4))
        bhi = jnp.tile(b_ref[:, FH:NHID], (1, 4))
        hlo = jnp.maximum(p_ref[0] + blo, 0.0)
        hhi = jnp.maximum(p_ref[1] + bhi, 0.0)
        o_ref[...] = (
            jnp.dot(hlo, wlo_ref[...], preferred_element_type=jnp.float32)
            + jnp.dot(hhi, whi_ref[...], preferred_element_type=jnp.float32))

    return pl.pallas_call(
        body,
        grid=(2,),
        in_specs=[pl.BlockSpec((2, blk, 128), lambda i: (0, i, 0)),
                  pl.BlockSpec((1, NHID), lambda i: (0, 0)),
                  pl.BlockSpec((128, 4 * NCLASS), lambda i: (0, 0)),
                  pl.BlockSpec((128, 4 * NCLASS), lambda i: (0, 0))],
        out_specs=pl.BlockSpec((blk, 4 * NCLASS), lambda i: (i, 0)),
        out_shape=jax.ShapeDtypeStruct((vin, 4 * NCLASS), jnp.float32),
    )(pv, b1, w2lo4, w2hi4)


def _tc_tail(pv, b2t, sub_fea, fc1_wT, fc1_b, att_W, att_b, att_a):
    nreal = N * NCLASS // 128

    def body(p_ref, b2_ref, sub_ref, fwT_ref, fb_ref, aW_ref, ab_ref,
             aa_ref, o_ref):
        b2t = jnp.tile(b2_ref[...], (1, 8))
        h = jnp.maximum(p_ref[0, :nreal, :] + p_ref[1, :nreal, :]
                        + b2t, 0.0)
        s128 = jnp.sum(h, axis=0, keepdims=True)
        mean = (s128[:, 0:16] + s128[:, 16:32] + s128[:, 32:48]
                + s128[:, 48:64] + s128[:, 64:80] + s128[:, 80:96]
                + s128[:, 96:112] + s128[:, 112:128]) * (1.0 / N)
        pooled = 1.0507009873554805 * jnp.where(
            mean > 0, mean, 1.6732632423543772 * (jnp.exp(mean) - 1.0))
        x_ext = jnp.dot(sub_ref[...], fwT_ref[...],
                        preferred_element_type=jnp.float32) + fb_ref[...]
        xc = jnp.concatenate([pooled, x_ext], axis=1)
        heads = []
        for hh in range(4):
            heads.append(jnp.dot(xc, aW_ref[hh],
                                 preferred_element_type=jnp.float32)
                         + ab_ref[hh:hh + 1])
        hm = jnp.concatenate(heads, axis=0)
        scores = jnp.sum(hm * aa_ref[...], axis=1, keepdims=True)
        mx = jnp.max(scores, axis=0, keepdims=True)
        ex = jnp.exp(scores - mx)
        alpha = ex / jnp.sum(ex, axis=0, keepdims=True)
        out = jnp.sum(alpha * hm, axis=0, keepdims=True)
        m2 = jnp.max(out, axis=1, keepdims=True)
        lse = jnp.log(jnp.sum(jnp.exp(out - m2), axis=1, keepdims=True)) + m2
        o_ref[...] = out - lse

    return pl.pallas_call(
        body,
        out_shape=jax.ShapeDtypeStruct((1, NCLASS), jnp.float32),
    )(pv, b2t, sub_fea, fc1_wT, fc1_b, att_W, att_b, att_a)


def kernel(x, adj, sub_fea, W1, b1, W2, b2, fc1_w, fc1_b, att_W, att_b, att_a):
    adj2 = adj.reshape(2, NREAL, CH)
    padc = jnp.asarray(_PAD_NP)
    zeros = jnp.asarray(_ZERO_NP)

    w2lo4 = jax.scipy.linalg.block_diag(*([W2[:FH]] * 4))
    w2hi4 = jax.scipy.linalg.block_diag(*([W2[FH:]] * 4))
    support1 = _tc_matmul(x, W1)
    p1 = _spmm64(support1, adj2, padc, zeros)
    p1v = p1.reshape(NC, NPAD * FH // 128, 128)
    support2 = _tc_layer2(p1v, b1.reshape(1, -1), w2lo4, w2hi4)
    p2 = _spmm16(support2.reshape(NPAD, NCLASS), adj2, padc, zeros)
    p2v = p2.reshape(NC, NPAD * NCLASS // 128, 128)
    return _tc_tail(p2v, b2.reshape(1, -1), sub_fea,
                    fc1_w.T, fc1_b.reshape(1, -1), att_W, att_b, att_a)
